# baseline (speedup 1.0000x reference)
import jax
import jax.numpy as jnp
from jax.experimental import pallas as pl

BATCH, SEQ = 4096, 20
NTOK = BATCH * SEQ
BLK = 256
NBLK = NTOK // BLK
V = 1000
D = 4


def _tc_body(idx_ref, embed_ref, w_ref, b_ref, out_ref):
    idx = idx_ref[0]
    iota_v = jax.lax.broadcasted_iota(jnp.int32, (V, BLK), 0)
    oh_t = (idx == iota_v).astype(jnp.float32)
    emb = jax.lax.dot_general(
        oh_t, embed_ref[...],
        dimension_numbers=(((0,), (0,)), ((), ())),
        preferred_element_type=jnp.float32)
    out = jax.lax.dot_general(
        emb, w_ref[...],
        dimension_numbers=(((1,), (1,)), ((), ())),
        preferred_element_type=jnp.float32)
    out_ref[...] = out + b_ref[...]


def kernel(x, embed, W, b):
    x3 = x.reshape(NBLK, 1, BLK).astype(jnp.int32)
    b2 = b.reshape(1, V)
    out = pl.pallas_call(
        _tc_body,
        grid=(NBLK,),
        in_specs=[
            pl.BlockSpec((1, 1, BLK), lambda i: (i, 0, 0)),
            pl.BlockSpec((V, D), lambda i: (0, 0)),
            pl.BlockSpec((V, D), lambda i: (0, 0)),
            pl.BlockSpec((1, V), lambda i: (0, 0)),
        ],
        out_specs=pl.BlockSpec((BLK, V), lambda i: (i, 0)),
        out_shape=jax.ShapeDtypeStruct((NTOK, V), jnp.float32),
    )(x3, embed, W, b2)
    return out.reshape(BATCH, SEQ, V)


# speedup vs baseline: 1.0073x; 1.0073x over previous
import functools

import jax
import jax.numpy as jnp
from jax import lax
from jax.experimental import pallas as pl
from jax.experimental.pallas import tpu as pltpu
from jax.experimental.pallas import tpu_sc as plsc

BATCH, SEQ = 4096, 20
NTOK = BATCH * SEQ
V = 1000
D = 4

NC, NS = 2, 16
NW = NC * NS
TPW = NTOK // NW
CHUNK = 40
NCHUNK = TPW // CHUNK


def _table_body(embed_ref, w_ref, b_ref, table_ref):
    table_ref[...] = lax.dot_general(
        embed_ref[...], w_ref[...],
        dimension_numbers=(((1,), (1,)), ((), ())),
        preferred_element_type=jnp.float32) + b_ref[...]


def _sc_gather_body(idx_hbm, table_hbm, out_hbm, idx_v, rows_v, sem):
    wid = lax.axis_index("s") * NC + lax.axis_index("c")
    base = wid * TPW
    pltpu.sync_copy(idx_hbm.at[pl.ds(base, TPW)], idx_v)

    def step(g, carry):
        off = g * CHUNK
        pltpu.async_copy(
            table_hbm.at[idx_v.at[pl.ds(off, CHUNK)]], rows_v, sem).wait()
        pltpu.sync_copy(rows_v, out_hbm.at[pl.ds(base + off, CHUNK)])
        return carry

    lax.fori_loop(0, NCHUNK, step, 0)


@functools.partial(
    pl.kernel,
    out_type=jax.ShapeDtypeStruct((NTOK, V), jnp.float32),
    mesh=plsc.VectorSubcoreMesh(core_axis_name="c", subcore_axis_name="s"),
    compiler_params=pltpu.CompilerParams(use_tc_tiling_on_sc=False),
    scratch_types=[
        pltpu.VMEM((TPW,), jnp.int32),
        pltpu.VMEM((CHUNK, V), jnp.float32),
        pltpu.SemaphoreType.DMA,
    ],
)
def _sc_gather(idx_hbm, table_hbm, out_hbm, idx_v, rows_v, sem):
    _sc_gather_body(idx_hbm, table_hbm, out_hbm, idx_v, rows_v, sem)


def kernel(x, embed, W, b):
    table = pl.pallas_call(
        _table_body,
        out_shape=jax.ShapeDtypeStruct((V, V), jnp.float32),
    )(embed, W, b.reshape(1, V))
    out = _sc_gather(x.reshape(NTOK).astype(jnp.int32), table)
    return out.reshape(BATCH, SEQ, V)
